# baseline (speedup 1.0000x reference)
import functools

import jax
import jax.numpy as jnp
from jax import lax
from jax.experimental import pallas as pl
from jax.experimental.pallas import tpu as pltpu
from jax.experimental.pallas import tpu_sc as plsc

VOCAB = 1000000
DIM = 32
B = 4096
L = 200

_INFO = plsc.get_sparse_core_info()
NC = _INFO.num_cores
NS = _INFO.num_subcores
NW = NC * NS
BW = B // NW
NBUF = 2
DH = DIM // 8


def _make_sc_gather():
  mesh = plsc.VectorSubcoreMesh(core_axis_name="c", subcore_axis_name="s")

  @functools.partial(
      pl.kernel,
      mesh=mesh,
      compiler_params=pltpu.CompilerParams(use_tc_tiling_on_sc=False,
                                           needs_layout_passes=False),
      out_type=jax.ShapeDtypeStruct((L, DH, B // 128, 8, 128), jnp.float32),
      scratch_types=[
          pltpu.VMEM((BW, L), jnp.int32),
          pltpu.VMEM((NBUF, BW), jnp.int32),
          pltpu.VMEM((NBUF, BW), jnp.int32),
          pltpu.VMEM((NBUF, BW, 128), jnp.float32),
          pltpu.VMEM((NBUF, DH, 8, 128), jnp.float32),
          pltpu.SemaphoreType.DMA((NBUF,)),
          pltpu.SemaphoreType.DMA((NBUF,)),
      ],
  )
  def gather_kernel(idx_hbm, table4_hbm, out5_hbm, idx_v, widx_v, qoff_v,
                    wide_v, tbuf, gsem, osem):
    wid = lax.axis_index("s") * NC + lax.axis_index("c")
    b0 = wid * BW

    pltpu.sync_copy(idx_hbm.at[pl.ds(b0, BW)], idx_v)

    iota = lax.iota(jnp.int32, 16)

    def prep_and_fire_gather(l, s):
      lv = jnp.zeros((16,), jnp.int32) + l
      for gb in range(8):
        v = plsc.load_gather(idx_v, [iota + gb * 16, lv])
        widx_v[s, pl.ds(gb * 16, 16)] = lax.shift_right_logical(v, 2)
        qoff_v[s, pl.ds(gb * 16, 16)] = lax.shift_left(v & 3, 5)
      pltpu.async_copy(table4_hbm.at[widx_v.at[s]], wide_v.at[s], gsem.at[s])

    def wait_gather(s):
      pltpu.make_async_copy(table4_hbm.at[widx_v.at[s]], wide_v.at[s],
                            gsem.at[s]).wait()

    def transpose(s):
      for gb in range(8):
        q = qoff_v[s, pl.ds(gb * 16, 16)]
        srcb = iota + gb * 16
        for dh in range(DH):
          for dl in range(8):
            v = plsc.load_gather(wide_v.at[s], [srcb, q + (dh * 8 + dl)])
            tbuf[s, dh, dl, pl.ds(gb * 16, 16)] = v

    def fire_out(l, s):
      for dh in range(DH):
        pltpu.async_copy(tbuf.at[s].at[dh], out5_hbm.at[l].at[dh].at[wid],
                         osem.at[s])

    def wait_out(l, s):
      for dh in range(DH):
        pltpu.make_async_copy(tbuf.at[s].at[dh], out5_hbm.at[l].at[dh].at[wid],
                              osem.at[s]).wait()

    for s in range(NBUF):
      prep_and_fire_gather(s, s)

    def body(l0, carry):
      for s in range(NBUF):
        l = l0 + s
        wait_gather(s)

        @pl.when(l >= NBUF)
        def _():
          wait_out(l, s)

        transpose(s)
        fire_out(l, s)

        @pl.when(l + NBUF < L)
        def _():
          prep_and_fire_gather(l + NBUF, s)
      return carry

    lax.fori_loop(0, L // NBUF, lambda i, c: body(i * NBUF, c), 0,
                  unroll=False)

    for s in range(NBUF):
      wait_out(L - NBUF + s, s)

  return gather_kernel


_sc_gather = _make_sc_gather()


@jax.jit
def kernel(inputs, table):
  out5 = _sc_gather(inputs.astype(jnp.int32), table.reshape(VOCAB // 4, 128))
  return out5.transpose(2, 4, 0, 1, 3).reshape(B, L, DIM)
